# baseline (device time: 211196 ns/iter reference)
import jax
import jax.numpy as jnp
from jax import lax
from jax.experimental import pallas as pl
from jax.experimental.pallas import tpu as pltpu


def kernel(dy, W):
    partial = lax.dot_general(
        dy.astype(jnp.bfloat16),
        W.astype(jnp.bfloat16),
        dimension_numbers=(((1,), (1,)), ((), ())),
        preferred_element_type=jnp.float32,
    )

    m, n = partial.shape

    def body(p_ref, out_ref, send_ref, recv_ref, send_sem, recv_sem):
        my_x = lax.axis_index("x")
        my_y = lax.axis_index("y")
        my_z = lax.axis_index("z")
        partner = (my_x, 1 - my_y, my_z)

        barrier_sem = pltpu.get_barrier_semaphore()
        pl.semaphore_signal(
            barrier_sem, inc=1,
            device_id=partner, device_id_type=pl.DeviceIdType.MESH,
        )
        pl.semaphore_wait(barrier_sem, 1)

        send_ref[...] = p_ref[...].astype(jnp.bfloat16)
        rdma = pltpu.make_async_remote_copy(
            src_ref=send_ref,
            dst_ref=recv_ref,
            send_sem=send_sem,
            recv_sem=recv_sem,
            device_id=partner,
            device_id_type=pl.DeviceIdType.MESH,
        )
        rdma.start()
        rdma.wait()

        out_ref[...] = p_ref[...] + recv_ref[...].astype(jnp.float32)

    return pl.pallas_call(
        body,
        out_shape=jax.ShapeDtypeStruct((m, n), jnp.float32),
        in_specs=[pl.BlockSpec(memory_space=pltpu.VMEM)],
        out_specs=pl.BlockSpec(memory_space=pltpu.VMEM),
        scratch_shapes=[
            pltpu.VMEM((m, n), jnp.bfloat16),
            pltpu.VMEM((m, n), jnp.bfloat16),
            pltpu.SemaphoreType.DMA,
            pltpu.SemaphoreType.DMA,
        ],
        compiler_params=pltpu.CompilerParams(collective_id=0),
    )(partial)


# device time: 102014 ns/iter; 2.0703x vs baseline; 2.0703x over previous
import functools

import jax
import jax.numpy as jnp
from jax import lax
from jax.experimental import pallas as pl
from jax.experimental.pallas import tpu as pltpu

N_RING = 8
TILE_M = 2048 // N_RING
N_FWD = 4
N_BWD = 3


def _ring_pos(x, z):
    return jnp.where(x == 0, z, (N_RING - 1) - z)


def _coords_of(rp):
    x = (rp >= 4).astype(rp.dtype)
    z = jnp.where(rp < 4, rp, (N_RING - 1) - rp)
    return x, z


def kernel(dy, W):
    my_x = lax.axis_index("x")
    my_z = lax.axis_index("z")
    pos = _ring_pos(my_x, my_z)

    dy_tile = lax.dynamic_slice_in_dim(dy, pos * TILE_M, TILE_M, axis=0)
    partial = lax.dot_general(
        dy_tile.astype(jnp.bfloat16),
        W.astype(jnp.bfloat16),
        dimension_numbers=(((1,), (1,)), ((), ())),
        preferred_element_type=jnp.float32,
    )

    n = partial.shape[1]

    def body(p_ref, out_ref,
             y_send, y_recv, red_bf, f_recv, b_recv,
             y_send_sem, y_recv_sem,
             f_send_sems, f_recv_sems, b_send_sems, b_recv_sems):
        x = lax.axis_index("x")
        y = lax.axis_index("y")
        z = lax.axis_index("z")
        rp = _ring_pos(x, z)
        rx, rz = _coords_of((rp + 1) % N_RING)
        lx, lz = _coords_of((rp - 1) % N_RING)
        right = (rx, y, rz)
        left = (lx, y, lz)
        partner = (x, 1 - y, z)

        barrier_sem = pltpu.get_barrier_semaphore()
        for nbr in (left, right, partner):
            pl.semaphore_signal(
                barrier_sem, inc=1,
                device_id=nbr, device_id_type=pl.DeviceIdType.MESH,
            )
        pl.semaphore_wait(barrier_sem, 3)

        y_send[...] = p_ref[...].astype(jnp.bfloat16)
        y_rdma = pltpu.make_async_remote_copy(
            src_ref=y_send, dst_ref=y_recv,
            send_sem=y_send_sem, recv_sem=y_recv_sem,
            device_id=partner, device_id_type=pl.DeviceIdType.MESH,
        )
        y_rdma.start()
        y_rdma.wait_recv()

        red_f32 = p_ref[...] + y_recv[...].astype(jnp.float32)
        out_ref[pl.ds(rp * TILE_M, TILE_M), :] = red_f32
        red_bf[...] = red_f32.astype(jnp.bfloat16)

        f_rdmas = []
        b_rdmas = []

        def mk(src, dst, ssem, rsem, dev):
            return pltpu.make_async_remote_copy(
                src_ref=src, dst_ref=dst, send_sem=ssem, recv_sem=rsem,
                device_id=dev, device_id_type=pl.DeviceIdType.MESH,
            )

        f_rdmas.append(mk(red_bf, f_recv.at[0],
                          f_send_sems.at[0], f_recv_sems.at[0], right))
        f_rdmas[0].start()
        b_rdmas.append(mk(red_bf, b_recv.at[0],
                          b_send_sems.at[0], b_recv_sems.at[0], left))
        b_rdmas[0].start()

        for h in range(N_FWD):
            f_rdmas[h].wait_recv()
            if h + 1 < N_FWD:
                nxt = mk(f_recv.at[h], f_recv.at[h + 1],
                         f_send_sems.at[h + 1], f_recv_sems.at[h + 1], right)
                nxt.start()
                f_rdmas.append(nxt)
            origin = (rp - 1 - h) % N_RING
            out_ref[pl.ds(origin * TILE_M, TILE_M), :] = (
                f_recv[h].astype(jnp.float32))

            if h < N_BWD:
                b_rdmas[h].wait_recv()
                if h + 1 < N_BWD:
                    nxt = mk(b_recv.at[h], b_recv.at[h + 1],
                             b_send_sems.at[h + 1], b_recv_sems.at[h + 1],
                             left)
                    nxt.start()
                    b_rdmas.append(nxt)
                origin = (rp + 1 + h) % N_RING
                out_ref[pl.ds(origin * TILE_M, TILE_M), :] = (
                    b_recv[h].astype(jnp.float32))

        y_rdma.wait_send()
        for r in f_rdmas + b_rdmas:
            r.wait_send()

        @functools.partial(pl.run_scoped,
                           exit_sem=pltpu.SemaphoreType.REGULAR)
        def _(exit_sem):
            for nbr in (left, right, partner):
                pl.semaphore_signal(
                    exit_sem, inc=1,
                    device_id=nbr, device_id_type=pl.DeviceIdType.MESH,
                )
            pl.semaphore_wait(exit_sem, 3)

    return pl.pallas_call(
        body,
        out_shape=jax.ShapeDtypeStruct((2048, n), jnp.float32),
        in_specs=[pl.BlockSpec(memory_space=pltpu.VMEM)],
        out_specs=pl.BlockSpec(memory_space=pltpu.VMEM),
        scratch_shapes=[
            pltpu.VMEM((TILE_M, n), jnp.bfloat16),
            pltpu.VMEM((TILE_M, n), jnp.bfloat16),
            pltpu.VMEM((TILE_M, n), jnp.bfloat16),
            pltpu.VMEM((N_FWD, TILE_M, n), jnp.bfloat16),
            pltpu.VMEM((N_BWD, TILE_M, n), jnp.bfloat16),
            pltpu.SemaphoreType.DMA,
            pltpu.SemaphoreType.DMA,
            pltpu.SemaphoreType.DMA((N_FWD,)),
            pltpu.SemaphoreType.DMA((N_FWD,)),
            pltpu.SemaphoreType.DMA((N_BWD,)),
            pltpu.SemaphoreType.DMA((N_BWD,)),
        ],
        compiler_params=pltpu.CompilerParams(collective_id=0),
    )(partial)


# device time: 90630 ns/iter; 2.3303x vs baseline; 1.1256x over previous
import functools

import jax
import jax.numpy as jnp
from jax import lax
from jax.experimental import pallas as pl
from jax.experimental.pallas import tpu as pltpu

N_RING = 8
TILE_M = 2048 // N_RING
N_FWD = 4
N_BWD = 3
N_CHUNK = 2


def _ring_pos(x, z):
    return jnp.where(x == 0, z, (N_RING - 1) - z)


def _coords_of(rp):
    x = (rp >= 4).astype(rp.dtype)
    z = jnp.where(rp < 4, rp, (N_RING - 1) - rp)
    return x, z


def kernel(dy, W):
    my_x = lax.axis_index("x")
    my_z = lax.axis_index("z")
    pos = _ring_pos(my_x, my_z)

    dy_tile = lax.dynamic_slice_in_dim(dy, pos * TILE_M, TILE_M, axis=0)
    partial = lax.dot_general(
        dy_tile.astype(jnp.bfloat16),
        W.astype(jnp.bfloat16),
        dimension_numbers=(((1,), (1,)), ((), ())),
        preferred_element_type=jnp.float32,
    )

    n = partial.shape[1]
    nc = n // N_CHUNK

    def body(p_ref, out_ref,
             y_send, y_recv, red_bf, f_recv, b_recv,
             y_send_sems, y_recv_sems,
             f_send_sems, f_recv_sems, b_send_sems, b_recv_sems):
        x = lax.axis_index("x")
        y = lax.axis_index("y")
        z = lax.axis_index("z")
        rp = _ring_pos(x, z)
        rx, rz = _coords_of((rp + 1) % N_RING)
        lx, lz = _coords_of((rp - 1) % N_RING)
        right = (rx, y, rz)
        left = (lx, y, lz)
        partner = (x, 1 - y, z)

        barrier_sem = pltpu.get_barrier_semaphore()
        for nbr in (left, right, partner):
            pl.semaphore_signal(
                barrier_sem, inc=1,
                device_id=nbr, device_id_type=pl.DeviceIdType.MESH,
            )
        pl.semaphore_wait(barrier_sem, 3)

        def mk(src, dst, ssem, rsem, dev):
            return pltpu.make_async_remote_copy(
                src_ref=src, dst_ref=dst, send_sem=ssem, recv_sem=rsem,
                device_id=dev, device_id_type=pl.DeviceIdType.MESH,
            )

        y_rdmas = []
        for c in range(N_CHUNK):
            y_send[c] = p_ref[:, pl.ds(c * nc, nc)].astype(jnp.bfloat16)
            r = mk(y_send.at[c], y_recv.at[c],
                   y_send_sems.at[c], y_recv_sems.at[c], partner)
            r.start()
            y_rdmas.append(r)

        f_rdmas = [[None] * N_CHUNK for _ in range(N_FWD)]
        b_rdmas = [[None] * N_CHUNK for _ in range(N_BWD)]

        for c in range(N_CHUNK):
            y_rdmas[c].wait_recv()
            red_f32 = (p_ref[:, pl.ds(c * nc, nc)]
                       + y_recv[c].astype(jnp.float32))
            out_ref[pl.ds(rp * TILE_M, TILE_M), pl.ds(c * nc, nc)] = red_f32
            red_bf[c] = red_f32.astype(jnp.bfloat16)
            fr = mk(red_bf.at[c], f_recv.at[0, c],
                    f_send_sems.at[0, c], f_recv_sems.at[0, c], right)
            fr.start()
            f_rdmas[0][c] = fr
            br = mk(red_bf.at[c], b_recv.at[0, c],
                    b_send_sems.at[0, c], b_recv_sems.at[0, c], left)
            br.start()
            b_rdmas[0][c] = br

        for h in range(N_FWD):
            f_origin = (rp - 1 - h) % N_RING
            b_origin = (rp + 1 + h) % N_RING
            for c in range(N_CHUNK):
                f_rdmas[h][c].wait_recv()
                if h + 1 < N_FWD:
                    nxt = mk(f_recv.at[h, c], f_recv.at[h + 1, c],
                             f_send_sems.at[h + 1, c],
                             f_recv_sems.at[h + 1, c], right)
                    nxt.start()
                    f_rdmas[h + 1][c] = nxt
                out_ref[pl.ds(f_origin * TILE_M, TILE_M),
                        pl.ds(c * nc, nc)] = f_recv[h, c].astype(jnp.float32)

                if h < N_BWD:
                    b_rdmas[h][c].wait_recv()
                    if h + 1 < N_BWD:
                        nxt = mk(b_recv.at[h, c], b_recv.at[h + 1, c],
                                 b_send_sems.at[h + 1, c],
                                 b_recv_sems.at[h + 1, c], left)
                        nxt.start()
                        b_rdmas[h + 1][c] = nxt
                    out_ref[pl.ds(b_origin * TILE_M, TILE_M),
                            pl.ds(c * nc, nc)] = (
                        b_recv[h, c].astype(jnp.float32))

        for r in y_rdmas:
            r.wait_send()
        for row in f_rdmas + b_rdmas:
            for r in row:
                r.wait_send()

        @functools.partial(pl.run_scoped,
                           exit_sem=pltpu.SemaphoreType.REGULAR)
        def _(exit_sem):
            for nbr in (left, right, partner):
                pl.semaphore_signal(
                    exit_sem, inc=1,
                    device_id=nbr, device_id_type=pl.DeviceIdType.MESH,
                )
            pl.semaphore_wait(exit_sem, 3)

    return pl.pallas_call(
        body,
        out_shape=jax.ShapeDtypeStruct((2048, n), jnp.float32),
        in_specs=[pl.BlockSpec(memory_space=pltpu.VMEM)],
        out_specs=pl.BlockSpec(memory_space=pltpu.VMEM),
        scratch_shapes=[
            pltpu.VMEM((N_CHUNK, TILE_M, nc), jnp.bfloat16),
            pltpu.VMEM((N_CHUNK, TILE_M, nc), jnp.bfloat16),
            pltpu.VMEM((N_CHUNK, TILE_M, nc), jnp.bfloat16),
            pltpu.VMEM((N_FWD, N_CHUNK, TILE_M, nc), jnp.bfloat16),
            pltpu.VMEM((N_BWD, N_CHUNK, TILE_M, nc), jnp.bfloat16),
            pltpu.SemaphoreType.DMA((N_CHUNK,)),
            pltpu.SemaphoreType.DMA((N_CHUNK,)),
            pltpu.SemaphoreType.DMA((N_FWD, N_CHUNK)),
            pltpu.SemaphoreType.DMA((N_FWD, N_CHUNK)),
            pltpu.SemaphoreType.DMA((N_BWD, N_CHUNK)),
            pltpu.SemaphoreType.DMA((N_BWD, N_CHUNK)),
        ],
        compiler_params=pltpu.CompilerParams(collective_id=0),
    )(partial)


# device time: 89494 ns/iter; 2.3599x vs baseline; 1.0127x over previous
import functools

import jax
import jax.numpy as jnp
from jax import lax
from jax.experimental import pallas as pl
from jax.experimental.pallas import tpu as pltpu

N_RING = 8
TILE_M = 2048 // N_RING
N_FWD = 4
N_BWD = 3
N_CHUNK = 2


def _ring_pos(x, z):
    return jnp.where(x == 0, z, (N_RING - 1) - z)


def _coords_of(rp):
    x = (rp >= 4).astype(rp.dtype)
    z = jnp.where(rp < 4, rp, (N_RING - 1) - rp)
    return x, z


def kernel(dy, W):
    my_x = lax.axis_index("x")
    my_z = lax.axis_index("z")
    pos = _ring_pos(my_x, my_z)

    dy_tile = lax.dynamic_slice_in_dim(dy, pos * TILE_M, TILE_M, axis=0)
    partial = lax.dot_general(
        dy_tile,
        W,
        dimension_numbers=(((1,), (1,)), ((), ())),
        preferred_element_type=jnp.float32,
        precision=lax.Precision.DEFAULT,
    )

    n = partial.shape[1]
    nc = n // N_CHUNK

    def body(p_ref, out_ref,
             y_send, y_recv, red_bf, f_recv, b_recv,
             y_send_sems, y_recv_sems,
             f_send_sems, f_recv_sems, b_send_sems, b_recv_sems):
        x = lax.axis_index("x")
        y = lax.axis_index("y")
        z = lax.axis_index("z")
        rp = _ring_pos(x, z)
        rx, rz = _coords_of((rp + 1) % N_RING)
        lx, lz = _coords_of((rp - 1) % N_RING)
        right = (rx, y, rz)
        left = (lx, y, lz)
        partner = (x, 1 - y, z)

        barrier_sem = pltpu.get_barrier_semaphore()
        for nbr in (left, right, partner):
            pl.semaphore_signal(
                barrier_sem, inc=1,
                device_id=nbr, device_id_type=pl.DeviceIdType.MESH,
            )
        pl.semaphore_wait(barrier_sem, 3)

        def mk(src, dst, ssem, rsem, dev):
            return pltpu.make_async_remote_copy(
                src_ref=src, dst_ref=dst, send_sem=ssem, recv_sem=rsem,
                device_id=dev, device_id_type=pl.DeviceIdType.MESH,
            )

        y_rdmas = []
        for c in range(N_CHUNK):
            y_send[c] = p_ref[:, pl.ds(c * nc, nc)].astype(jnp.bfloat16)
            r = mk(y_send.at[c], y_recv.at[c],
                   y_send_sems.at[c], y_recv_sems.at[c], partner)
            r.start()
            y_rdmas.append(r)

        f_rdmas = [[None] * N_CHUNK for _ in range(N_FWD)]
        b_rdmas = [[None] * N_CHUNK for _ in range(N_BWD)]

        for c in range(N_CHUNK):
            y_rdmas[c].wait_recv()
            red_f32 = (p_ref[:, pl.ds(c * nc, nc)]
                       + y_recv[c].astype(jnp.float32))
            out_ref[pl.ds(rp * TILE_M, TILE_M), pl.ds(c * nc, nc)] = red_f32
            red_bf[c] = red_f32.astype(jnp.bfloat16)
            fr = mk(red_bf.at[c], f_recv.at[0, c],
                    f_send_sems.at[0, c], f_recv_sems.at[0, c], right)
            fr.start()
            f_rdmas[0][c] = fr
            br = mk(red_bf.at[c], b_recv.at[0, c],
                    b_send_sems.at[0, c], b_recv_sems.at[0, c], left)
            br.start()
            b_rdmas[0][c] = br

        for h in range(N_FWD):
            f_origin = (rp - 1 - h) % N_RING
            b_origin = (rp + 1 + h) % N_RING
            for c in range(N_CHUNK):
                f_rdmas[h][c].wait_recv()
                if h + 1 < N_FWD:
                    nxt = mk(f_recv.at[h, c], f_recv.at[h + 1, c],
                             f_send_sems.at[h + 1, c],
                             f_recv_sems.at[h + 1, c], right)
                    nxt.start()
                    f_rdmas[h + 1][c] = nxt
                out_ref[pl.ds(f_origin * TILE_M, TILE_M),
                        pl.ds(c * nc, nc)] = f_recv[h, c].astype(jnp.float32)

                if h < N_BWD:
                    b_rdmas[h][c].wait_recv()
                    if h + 1 < N_BWD:
                        nxt = mk(b_recv.at[h, c], b_recv.at[h + 1, c],
                                 b_send_sems.at[h + 1, c],
                                 b_recv_sems.at[h + 1, c], left)
                        nxt.start()
                        b_rdmas[h + 1][c] = nxt
                    out_ref[pl.ds(b_origin * TILE_M, TILE_M),
                            pl.ds(c * nc, nc)] = (
                        b_recv[h, c].astype(jnp.float32))

        for r in y_rdmas:
            r.wait_send()
        for row in f_rdmas + b_rdmas:
            for r in row:
                r.wait_send()

        @functools.partial(pl.run_scoped,
                           exit_sem=pltpu.SemaphoreType.REGULAR)
        def _(exit_sem):
            for nbr in (left, right, partner):
                pl.semaphore_signal(
                    exit_sem, inc=1,
                    device_id=nbr, device_id_type=pl.DeviceIdType.MESH,
                )
            pl.semaphore_wait(exit_sem, 3)

    return pl.pallas_call(
        body,
        out_shape=jax.ShapeDtypeStruct((2048, n), jnp.float32),
        in_specs=[pl.BlockSpec(memory_space=pltpu.VMEM)],
        out_specs=pl.BlockSpec(memory_space=pltpu.VMEM),
        scratch_shapes=[
            pltpu.VMEM((N_CHUNK, TILE_M, nc), jnp.bfloat16),
            pltpu.VMEM((N_CHUNK, TILE_M, nc), jnp.bfloat16),
            pltpu.VMEM((N_CHUNK, TILE_M, nc), jnp.bfloat16),
            pltpu.VMEM((N_FWD, N_CHUNK, TILE_M, nc), jnp.bfloat16),
            pltpu.VMEM((N_BWD, N_CHUNK, TILE_M, nc), jnp.bfloat16),
            pltpu.SemaphoreType.DMA((N_CHUNK,)),
            pltpu.SemaphoreType.DMA((N_CHUNK,)),
            pltpu.SemaphoreType.DMA((N_FWD, N_CHUNK)),
            pltpu.SemaphoreType.DMA((N_FWD, N_CHUNK)),
            pltpu.SemaphoreType.DMA((N_BWD, N_CHUNK)),
            pltpu.SemaphoreType.DMA((N_BWD, N_CHUNK)),
        ],
        compiler_params=pltpu.CompilerParams(collective_id=0),
    )(partial)
